# baseline (speedup 1.0000x reference)
import jax
import jax.numpy as jnp
from jax import lax
from jax.experimental import pallas as pl
from jax.experimental.pallas import tpu as pltpu
from jax.experimental.pallas import tpu_sc as plsc

N_B = 4096
N_S = 50
N_D = 128
N_ROWS = N_B * N_S

NC, NS = 2, 16
NW = NC * NS
ROWS_PER_W = N_ROWS // NW
CHUNK = 128
N_CHUNKS = ROWS_PER_W // CHUNK
NBUF = 5
GROUPS = N_CHUNKS // NBUF


def _emb_body(idx_hbm, w_hbm, out_hbm, idx_v, rows_v, gsem, osem):
    wid = lax.axis_index("s") * NC + lax.axis_index("c")
    base = wid * ROWS_PER_W

    pltpu.sync_copy(idx_hbm.at[wid], idx_v)

    def gather_desc(c, b):
        return pltpu.make_async_copy(w_hbm.at[idx_v.at[c]], rows_v.at[b],
                                     gsem.at[b])

    def out_desc(c, b):
        dst = out_hbm.at[pl.ds(base + c * CHUNK, CHUNK)]
        return pltpu.make_async_copy(rows_v.at[b], dst, osem.at[b])

    for b in range(NBUF):
        gather_desc(b, b).start()

    @pl.loop(0, GROUPS)
    def _group(g):
        c0 = g * NBUF
        for b in range(NBUF):
            gather_desc(c0 + b, b).wait()
            out_desc(c0 + b, b).start()

        @pl.when(g + 1 < GROUPS)
        def _():
            for b in range(NBUF):
                out_desc(c0 + b, b).wait()
                gather_desc(c0 + NBUF + b, b).start()

    last = (GROUPS - 1) * NBUF
    for b in range(NBUF):
        out_desc(last + b, b).wait()


_emb_call = pl.kernel(
    _emb_body,
    out_type=jax.ShapeDtypeStruct((N_ROWS, N_D), jnp.float32),
    mesh=plsc.VectorSubcoreMesh(core_axis_name="c", subcore_axis_name="s"),
    scratch_types=[
        pltpu.VMEM((N_CHUNKS, CHUNK), jnp.int32),
        pltpu.VMEM((NBUF, CHUNK, N_D), jnp.float32),
        pltpu.SemaphoreType.DMA((NBUF,)),
        pltpu.SemaphoreType.DMA((NBUF,)),
    ],
    compiler_params=pltpu.CompilerParams(use_tc_tiling_on_sc=True),
)


@jax.jit
def kernel(input_, weight):
    idx = input_.reshape(NW, N_CHUNKS, CHUNK)
    out = _emb_call(idx, weight)
    return out.reshape(N_B, N_S, N_D)


# speedup vs baseline: 1.5937x; 1.5937x over previous
import jax
import jax.numpy as jnp
from jax import lax
from jax.experimental import pallas as pl
from jax.experimental.pallas import tpu as pltpu
from jax.experimental.pallas import tpu_sc as plsc

N_B = 4096
N_S = 50
N_D = 128

NC, NS = 2, 16
NW = NC * NS
B_PER_W = N_B // NW
NBUF = 4
GROUPS = B_PER_W // NBUF


def _emb_body(idx_hbm, w_hbm, out_hbm, idx_v, rows_v, gsem, osem):
    wid = lax.axis_index("s") * NC + lax.axis_index("c")
    base = wid * B_PER_W

    pltpu.sync_copy(idx_hbm.at[wid], idx_v)

    def gather_desc(c, b):
        return pltpu.make_async_copy(w_hbm.at[idx_v.at[c]], rows_v.at[b],
                                     gsem.at[b])

    def out_desc(c, b):
        return pltpu.make_async_copy(rows_v.at[b], out_hbm.at[base + c],
                                     osem.at[b])

    for b in range(NBUF):
        gather_desc(b, b).start()

    @pl.loop(0, GROUPS)
    def _group(g):
        c0 = g * NBUF
        for b in range(NBUF):
            gather_desc(c0 + b, b).wait()
            out_desc(c0 + b, b).start()

        @pl.when(g + 1 < GROUPS)
        def _():
            for b in range(NBUF):
                out_desc(c0 + b, b).wait()
                gather_desc(c0 + NBUF + b, b).start()

    last = (GROUPS - 1) * NBUF
    for b in range(NBUF):
        out_desc(last + b, b).wait()


_emb_call = pl.kernel(
    _emb_body,
    out_type=jax.ShapeDtypeStruct((N_B, N_S, N_D), jnp.float32),
    mesh=plsc.VectorSubcoreMesh(core_axis_name="c", subcore_axis_name="s"),
    scratch_types=[
        pltpu.VMEM((B_PER_W, N_S), jnp.int32),
        pltpu.VMEM((NBUF, N_S, N_D), jnp.float32),
        pltpu.SemaphoreType.DMA((NBUF,)),
        pltpu.SemaphoreType.DMA((NBUF,)),
    ],
    compiler_params=pltpu.CompilerParams(use_tc_tiling_on_sc=True),
)


@jax.jit
def kernel(input_, weight):
    idx = input_.reshape(NW, B_PER_W, N_S)
    return _emb_call(idx, weight)


# speedup vs baseline: 2.6079x; 1.6364x over previous
import jax
import jax.numpy as jnp
from jax import lax
from jax.experimental import pallas as pl
from jax.experimental.pallas import tpu as pltpu
from jax.experimental.pallas import tpu_sc as plsc

N_B = 4096
N_S = 50
N_D = 128

NC, NS = 2, 16
NW = NC * NS
B_PER_W = N_B // NW
NBUF = 4
GROUPS = B_PER_W // NBUF


def _emb_body(idx_hbm, w_hbm, out_hbm, idx_v, rows_v, table_sh, gsem, osem):
    sid = lax.axis_index("s")
    wid = sid * NC + lax.axis_index("c")
    base = wid * B_PER_W

    @pl.when(sid == 0)
    def _():
        pltpu.sync_copy(w_hbm, table_sh)

    pltpu.sync_copy(idx_hbm.at[wid], idx_v)
    plsc.subcore_barrier()

    def gather_desc(c, b):
        return pltpu.make_async_copy(table_sh.at[idx_v.at[c]], rows_v.at[b],
                                     gsem.at[b])

    def out_desc(c, b):
        return pltpu.make_async_copy(rows_v.at[b], out_hbm.at[base + c],
                                     osem.at[b])

    for b in range(NBUF):
        gather_desc(b, b).start()

    @pl.loop(0, GROUPS)
    def _group(g):
        c0 = g * NBUF
        for b in range(NBUF):
            gather_desc(c0 + b, b).wait()
            out_desc(c0 + b, b).start()

        @pl.when(g + 1 < GROUPS)
        def _():
            for b in range(NBUF):
                out_desc(c0 + b, b).wait()
                gather_desc(c0 + NBUF + b, b).start()

    last = (GROUPS - 1) * NBUF
    for b in range(NBUF):
        out_desc(last + b, b).wait()


_emb_call = pl.kernel(
    _emb_body,
    out_type=jax.ShapeDtypeStruct((N_B, N_S, N_D), jnp.float32),
    mesh=plsc.VectorSubcoreMesh(core_axis_name="c", subcore_axis_name="s"),
    scratch_types=[
        pltpu.VMEM((B_PER_W, N_S), jnp.int32),
        pltpu.VMEM((NBUF, N_S, N_D), jnp.float32),
        pltpu.VMEM_SHARED((1000, N_D), jnp.float32),
        pltpu.SemaphoreType.DMA((NBUF,)),
        pltpu.SemaphoreType.DMA((NBUF,)),
    ],
    compiler_params=pltpu.CompilerParams(use_tc_tiling_on_sc=True),
)


@jax.jit
def kernel(input_, weight):
    idx = input_.reshape(NW, B_PER_W, N_S)
    return _emb_call(idx, weight)
